# baseline (speedup 1.0000x reference)
import jax
import jax.numpy as jnp
from jax.experimental import pallas as pl
from jax.experimental.pallas import tpu as pltpu

_B, _S, _BULK, _TOWER = 8, 2048, 512, 128
_NUM_LEVELS = 16
_LN_EPS = 1e-6
_BLK = 512


def _layernorm_rows(h, g, b):
    mu = jnp.mean(h, axis=-1, keepdims=True)
    var = jnp.mean((h - mu) ** 2, axis=-1, keepdims=True)
    return (h - mu) / jnp.sqrt(var + _LN_EPS) * g + b


def _d8_nearest_t(y):
    r = jnp.round(y)
    err = y - r
    a = jnp.abs(err)
    m = jnp.max(a, axis=0, keepdims=True)
    ii = jax.lax.broadcasted_iota(jnp.int32, y.shape, 0)
    cand = jnp.where(a >= m, ii, 8)
    sel = jnp.min(cand, axis=0, keepdims=True)
    onehot = (ii == sel).astype(y.dtype)
    sign = jnp.where(err >= 0.0, 1.0, -1.0)
    odd = jnp.sum(r, axis=0, keepdims=True).astype(jnp.int32) & 1
    return r + sign * onehot * odd.astype(y.dtype)


def _e8_nearest_t(x):
    a = _d8_nearest_t(x)
    b = _d8_nearest_t(x - 0.5) + 0.5
    xa = x - a
    xb = x - b
    d = jnp.sum(xa * xa - xb * xb, axis=0, keepdims=True)
    return jnp.where(d <= 0.0, a, b)


def _block_kernel(
    u_ref, w_et, b_et, g_dt, be_dt, w_tb, b_tb, g_db, be_db,
    out_ref, codes_ref,
):
    u = u_ref[...].T
    quant = jnp.zeros_like(u)
    code_rows = []
    scale = 1.0
    for _ in range(_NUM_LEVELS):
        lat = _e8_nearest_t(u)
        code_rows.append((2.0 * lat).astype(jnp.int32))
        quant = quant + scale * lat
        u = 2.0 * (u - lat)
        scale = scale * 0.5

    codes_t = jnp.concatenate(code_rows, axis=0)
    codes_ref[...] = codes_t.T

    dt = jax.lax.dot_general(
        quant, w_et[...], (((0,), (0,)), ((), ())),
        preferred_element_type=jnp.float32,
    ) + b_et[...]
    dt = _layernorm_rows(dt, g_dt[...], be_dt[...])
    bulk = jnp.dot(dt, w_tb[...], preferred_element_type=jnp.float32) + b_tb[...]
    out_ref[...] = _layernorm_rows(bulk, g_db[...], be_db[...])


@jax.jit
def _run(x, params):
    n = _B * _S
    h = x.reshape(n, _BULK)

    tower = h @ params['W_bt'] + params['b_bt']
    mu = jnp.mean(tower, axis=-1, keepdims=True)
    var = jnp.mean((tower - mu) ** 2, axis=-1, keepdims=True)
    tower = (tower - mu) / jnp.sqrt(var + _LN_EPS) * params['g_enc_t'] + params['be_enc_t']
    e8c = tower @ params['W_te'] + params['b_te']
    rms = jnp.sqrt(jnp.mean(e8c ** 2, axis=-1, keepdims=True) + _LN_EPS)
    e8c = e8c / rms * params['rms_scale']

    def row2d(v):
        return v.reshape(1, -1)

    w_args = (
        params['W_et'], row2d(params['b_et']),
        row2d(params['g_dec_t']), row2d(params['be_dec_t']),
        params['W_tb'], row2d(params['b_tb']),
        row2d(params['g_dec_b']), row2d(params['be_dec_b']),
    )

    def full(a):
        return pl.BlockSpec(a.shape, lambda i: (0,) * a.ndim)

    in_specs = [pl.BlockSpec((_BLK, 8), lambda i: (i, 0))]
    in_specs += [full(a) for a in w_args]

    out, codes = pl.pallas_call(
        _block_kernel,
        grid=(n // _BLK,),
        in_specs=in_specs,
        out_specs=[
            pl.BlockSpec((_BLK, _BULK), lambda i: (i, 0)),
            pl.BlockSpec((_BLK, _NUM_LEVELS * 8), lambda i: (i, 0)),
        ],
        out_shape=[
            jax.ShapeDtypeStruct((n, _BULK), jnp.float32),
            jax.ShapeDtypeStruct((n, _NUM_LEVELS * 8), jnp.int32),
        ],
        compiler_params=pltpu.CompilerParams(
            dimension_semantics=("arbitrary",),
        ),
    )(e8c, *w_args)

    recon = out.reshape(_B, _S, _BULK)
    codes = codes.reshape(_B, _S, _NUM_LEVELS, 8)
    return recon, codes


def kernel(x, params):
    return _run(x, params)


# speedup vs baseline: 1.1012x; 1.1012x over previous
import jax
import jax.numpy as jnp
from jax.experimental import pallas as pl
from jax.experimental.pallas import tpu as pltpu

_B, _S, _BULK, _TOWER = 8, 2048, 512, 128
_NUM_LEVELS = 16
_LN_EPS = 1e-6
_BLK = 512


def _layernorm_rows(h, g, b):
    mu = jnp.mean(h, axis=-1, keepdims=True)
    var = jnp.mean((h - mu) ** 2, axis=-1, keepdims=True)
    return (h - mu) / jnp.sqrt(var + _LN_EPS) * g + b


def _d8_nearest_t(y):
    r = jnp.round(y)
    err = y - r
    a = jnp.abs(err)
    m = jnp.max(a, axis=0, keepdims=True)
    ii = jax.lax.broadcasted_iota(jnp.int32, y.shape, 0)
    cand = jnp.where(a >= m, ii, 8)
    sel = jnp.min(cand, axis=0, keepdims=True)
    onehot = (ii == sel).astype(y.dtype)
    sign = jnp.where(err >= 0.0, 1.0, -1.0)
    odd = jnp.sum(r, axis=0, keepdims=True).astype(jnp.int32) & 1
    return r + sign * onehot * odd.astype(y.dtype)


def _e8_nearest_t(x):
    a = _d8_nearest_t(x)
    b = _d8_nearest_t(x - 0.5) + 0.5
    xa = x - a
    xb = x - b
    d = jnp.sum(xa * xa - xb * xb, axis=0, keepdims=True)
    return jnp.where(d <= 0.0, a, b)


def _block_kernel(
    u_ref, w_et, b_et, g_dt, be_dt, w_tb, b_tb, g_db, be_db,
    out_ref, codes_ref,
):
    u = u_ref[...]
    quant = jnp.zeros_like(u)
    code_rows = []
    scale = 1.0
    for _ in range(_NUM_LEVELS):
        lat = _e8_nearest_t(u)
        code_rows.append((2.0 * lat).astype(jnp.int32))
        quant = quant + scale * lat
        u = 2.0 * (u - lat)
        scale = scale * 0.5

    codes_t = jnp.concatenate(code_rows, axis=0)
    codes_ref[...] = codes_t.T

    dt = jax.lax.dot_general(
        quant, w_et[...], (((0,), (0,)), ((), ())),
        preferred_element_type=jnp.float32,
    ) + b_et[...]
    dt = _layernorm_rows(dt, g_dt[...], be_dt[...])
    bulk = jnp.dot(dt, w_tb[...], preferred_element_type=jnp.float32) + b_tb[...]
    out_ref[...] = _layernorm_rows(bulk, g_db[...], be_db[...])


@jax.jit
def _run(x, params):
    n = _B * _S
    h = x.reshape(n, _BULK)

    tower = h @ params['W_bt'] + params['b_bt']
    mu = jnp.mean(tower, axis=-1, keepdims=True)
    var = jnp.mean((tower - mu) ** 2, axis=-1, keepdims=True)
    tower = (tower - mu) / jnp.sqrt(var + _LN_EPS) * params['g_enc_t'] + params['be_enc_t']
    e8c = tower @ params['W_te'] + params['b_te']
    rms = jnp.sqrt(jnp.mean(e8c ** 2, axis=-1, keepdims=True) + _LN_EPS)
    e8c = e8c / rms * params['rms_scale']

    u_t = e8c.T

    def row2d(v):
        return v.reshape(1, -1)

    w_args = (
        params['W_et'], row2d(params['b_et']),
        row2d(params['g_dec_t']), row2d(params['be_dec_t']),
        params['W_tb'], row2d(params['b_tb']),
        row2d(params['g_dec_b']), row2d(params['be_dec_b']),
    )

    def full(a):
        return pl.BlockSpec(a.shape, lambda i: (0,) * a.ndim)

    in_specs = [pl.BlockSpec((8, _BLK), lambda i: (0, i))]
    in_specs += [full(a) for a in w_args]

    out, codes = pl.pallas_call(
        _block_kernel,
        grid=(n // _BLK,),
        in_specs=in_specs,
        out_specs=[
            pl.BlockSpec((_BLK, _BULK), lambda i: (i, 0)),
            pl.BlockSpec((_BLK, _NUM_LEVELS * 8), lambda i: (i, 0)),
        ],
        out_shape=[
            jax.ShapeDtypeStruct((n, _BULK), jnp.float32),
            jax.ShapeDtypeStruct((n, _NUM_LEVELS * 8), jnp.int32),
        ],
        compiler_params=pltpu.CompilerParams(
            dimension_semantics=("arbitrary",),
        ),
    )(u_t, *w_args)

    recon = out.reshape(_B, _S, _BULK)
    codes = codes.reshape(_B, _S, _NUM_LEVELS, 8)
    return recon, codes


def kernel(x, params):
    return _run(x, params)


# speedup vs baseline: 1.1846x; 1.0758x over previous
import jax
import jax.numpy as jnp
from jax.experimental import pallas as pl
from jax.experimental.pallas import tpu as pltpu

_B, _S, _BULK, _TOWER = 8, 2048, 512, 128
_NUM_LEVELS = 16
_LN_EPS = 1e-6
_BLK = 1024


def _layernorm_rows(h, g, b):
    mu = jnp.mean(h, axis=-1, keepdims=True)
    var = jnp.mean((h - mu) ** 2, axis=-1, keepdims=True)
    return (h - mu) / jnp.sqrt(var + _LN_EPS) * g + b


def _d8_nearest_t(y):
    r = jnp.round(y)
    err = y - r
    a = jnp.abs(err)
    m = jnp.max(a, axis=0, keepdims=True)
    ii = jax.lax.broadcasted_iota(jnp.int32, y.shape, 0)
    cand = jnp.where(a >= m, ii, 8)
    sel = jnp.min(cand, axis=0, keepdims=True)
    onehot = (ii == sel).astype(y.dtype)
    sign = jnp.where(err >= 0.0, 1.0, -1.0)
    odd = jnp.sum(r, axis=0, keepdims=True).astype(jnp.int32) & 1
    return r + sign * onehot * odd.astype(y.dtype)


def _e8_nearest_t(x):
    a = _d8_nearest_t(x)
    b = _d8_nearest_t(x - 0.5) + 0.5
    xa = x - a
    xb = x - b
    d = jnp.sum(xa * xa - xb * xb, axis=0, keepdims=True)
    return jnp.where(d <= 0.0, a, b)


def _block_kernel(
    u_ref, w_et, b_et, g_dt, be_dt, w_tb, b_tb, g_db, be_db,
    out_ref, codes_ref,
):
    u = u_ref[...]
    quant = jnp.zeros_like(u)
    code_rows = []
    scale = 1.0
    for _ in range(_NUM_LEVELS):
        lat = _e8_nearest_t(u)
        code_rows.append((2.0 * lat).astype(jnp.int32))
        quant = quant + scale * lat
        u = 2.0 * (u - lat)
        scale = scale * 0.5

    codes_t = jnp.concatenate(code_rows, axis=0)
    codes_ref[...] = codes_t.T

    dt = jax.lax.dot_general(
        quant, w_et[...], (((0,), (0,)), ((), ())),
        preferred_element_type=jnp.float32,
    ) + b_et[...]
    dt = _layernorm_rows(dt, g_dt[...], be_dt[...])
    bulk = jnp.dot(dt, w_tb[...], preferred_element_type=jnp.float32) + b_tb[...]
    out_ref[...] = _layernorm_rows(bulk, g_db[...], be_db[...])


@jax.jit
def _run(x, params):
    n = _B * _S
    h = x.reshape(n, _BULK)

    tower = h @ params['W_bt'] + params['b_bt']
    mu = jnp.mean(tower, axis=-1, keepdims=True)
    var = jnp.mean((tower - mu) ** 2, axis=-1, keepdims=True)
    tower = (tower - mu) / jnp.sqrt(var + _LN_EPS) * params['g_enc_t'] + params['be_enc_t']
    e8c = tower @ params['W_te'] + params['b_te']
    rms = jnp.sqrt(jnp.mean(e8c ** 2, axis=-1, keepdims=True) + _LN_EPS)
    e8c = e8c / rms * params['rms_scale']

    u_t = e8c.T

    def row2d(v):
        return v.reshape(1, -1)

    w_args = (
        params['W_et'], row2d(params['b_et']),
        row2d(params['g_dec_t']), row2d(params['be_dec_t']),
        params['W_tb'], row2d(params['b_tb']),
        row2d(params['g_dec_b']), row2d(params['be_dec_b']),
    )

    def full(a):
        return pl.BlockSpec(a.shape, lambda i: (0,) * a.ndim)

    in_specs = [pl.BlockSpec((8, _BLK), lambda i: (0, i))]
    in_specs += [full(a) for a in w_args]

    out, codes = pl.pallas_call(
        _block_kernel,
        grid=(n // _BLK,),
        in_specs=in_specs,
        out_specs=[
            pl.BlockSpec((_BLK, _BULK), lambda i: (i, 0)),
            pl.BlockSpec((_BLK, _NUM_LEVELS * 8), lambda i: (i, 0)),
        ],
        out_shape=[
            jax.ShapeDtypeStruct((n, _BULK), jnp.float32),
            jax.ShapeDtypeStruct((n, _NUM_LEVELS * 8), jnp.int32),
        ],
        compiler_params=pltpu.CompilerParams(
            dimension_semantics=("parallel",),
        ),
    )(u_t, *w_args)

    recon = out.reshape(_B, _S, _BULK)
    codes = codes.reshape(_B, _S, _NUM_LEVELS, 8)
    return recon, codes


def kernel(x, params):
    return _run(x, params)
